# Initial kernel scaffold; baseline (speedup 1.0000x reference)
import jax
import jax.numpy as jnp
from jax.experimental import pallas as pl


def kernel(x, edge_index, batch, batch_size, c0_W1, c0_b1, c0_g, c0_be, c0_W2, c0_b2, c1_W1, c1_b1, c1_g, c1_be, c1_W2, c1_b2, c2_W1, c2_b1, c2_g, c2_be, c2_W2, c2_b2, f_W1, f_b1, f_W2, f_b2):
    raise NotImplementedError("write your pallas kernel here")


# speedup vs baseline: 2.8720x; 2.8720x over previous
import functools

import jax
import jax.numpy as jnp
from jax import lax
from jax.experimental import pallas as pl
from jax.experimental.pallas import tpu as pltpu
from jax.experimental.pallas import tpu_sc as plsc

N = 10000
B_GRAPHS = 64
HID = 256
OUT = 128

CH = 128
NSUB = 16
NBUF = 2
CPS = 160
SI = CPS // NBUF
CHUNKS = CPS * NSUB
E_PAD = CHUNKS * CH
RZ = 640
N_ACC = NSUB * RZ

R_BLK = 1000
GB = N // R_BLK


def _make_sc_agg(Hh, edge_split):
    mesh = plsc.VectorSubcoreMesh(core_axis_name="c", subcore_axis_name="s")

    def body(x2f, srcg, dstg, zrows, out, acc, srcb, dstb, rows, s0, s1):
        c = lax.axis_index("c")
        s = lax.axis_index("s")
        sems = (s0, s1)

        pltpu.sync_copy(zrows, rows.at[0])
        for k in range(RZ // CH):
            pltpu.sync_copy(rows.at[0], acc.at[pl.ds(s * RZ + k * CH, CH)])
        plsc.subcore_barrier()

        if edge_split:
            cbase = (c * NSUB + s) * (CPS // 2)
            iters = SI // 2
        else:
            cbase = s * CPS
            iters = SI

        def gbody(g, carry):
            gb = cbase + g * NBUF
            if edge_split:
                pltpu.sync_copy(srcg.at[pl.ds(gb, NBUF)], srcb)
            else:
                pltpu.sync_copy(srcg.at[c, pl.ds(gb, NBUF)], srcb)
            pltpu.sync_copy(dstg.at[pl.ds(gb, NBUF)], dstb)
            handles = [
                pltpu.async_copy(x2f.at[srcb.at[b]], rows.at[b], sems[b])
                for b in range(NBUF)
            ]
            for b in range(NBUF):
                handles[b].wait()
                pltpu.sync_copy(rows.at[b], acc.at[dstb.at[b]], add=True)
            return carry

        lax.fori_loop(0, iters, gbody, 0)
        plsc.subcore_barrier()

        for k in range(RZ // CH):
            pltpu.sync_copy(acc.at[pl.ds(s * RZ + k * CH, CH)], rows.at[0])
            pltpu.sync_copy(rows.at[0], out.at[c, pl.ds(s * RZ + k * CH, CH)])

    return pl.kernel(
        body,
        out_type=jax.ShapeDtypeStruct((2, N_ACC, Hh), jnp.float32),
        mesh=mesh,
        scratch_types=[
            pltpu.VMEM_SHARED((N_ACC, Hh), jnp.float32),
            pltpu.VMEM((NBUF, CH), jnp.int32),
            pltpu.VMEM((NBUF, CH), jnp.int32),
            pltpu.VMEM((NBUF, CH, Hh), jnp.float32),
            pltpu.SemaphoreType.DMA,
            pltpu.SemaphoreType.DMA,
        ],
    )


@functools.lru_cache(maxsize=None)
def _get_sc_agg(Hh, edge_split):
    return _make_sc_agg(Hh, edge_split)


def _mlp1a_body(x_ref, agg_ref, W1_ref, b1_ref, h_ref, stats_ref, acc_ref):
    i = pl.program_id(0)
    xa = x_ref[...] + agg_ref[0] + agg_ref[1]
    h = jnp.dot(xa, W1_ref[...], preferred_element_type=jnp.float32)
    h += b1_ref[...]
    h_ref[...] = h

    @pl.when(i == 0)
    def _():
        acc_ref[...] = jnp.zeros_like(acc_ref)

    acc_ref[0:1, :] += jnp.sum(h, axis=0, keepdims=True)
    acc_ref[1:2, :] += jnp.sum(h * h, axis=0, keepdims=True)

    @pl.when(i == GB - 1)
    def _():
        stats_ref[...] = acc_ref[...]


def _mlp1a(x, agg2, W1, b1):
    d_in = x.shape[1]
    return pl.pallas_call(
        _mlp1a_body,
        grid=(GB,),
        in_specs=[
            pl.BlockSpec((R_BLK, d_in), lambda i: (i, 0)),
            pl.BlockSpec((2, R_BLK, d_in), lambda i: (0, i, 0)),
            pl.BlockSpec(W1.shape, lambda i: (0, 0)),
            pl.BlockSpec((1, HID), lambda i: (0, 0)),
        ],
        out_specs=[
            pl.BlockSpec((R_BLK, HID), lambda i: (i, 0)),
            pl.BlockSpec((8, HID), lambda i: (0, 0)),
        ],
        out_shape=[
            jax.ShapeDtypeStruct((N, HID), jnp.float32),
            jax.ShapeDtypeStruct((8, HID), jnp.float32),
        ],
        scratch_shapes=[pltpu.VMEM((8, HID), jnp.float32)],
    )(x, agg2, W1, b1.reshape(1, HID))


def _mlp1_body(x2_ref, agg_ref, W1_ref, b1_ref, h_ref, stats_ref, acc_ref):
    i = pl.program_id(0)
    Hh = x2_ref.shape[2]
    lo = x2_ref[0] + agg_ref[0]
    hi = x2_ref[1] + agg_ref[1]
    h = jnp.dot(lo, W1_ref[:Hh, :], preferred_element_type=jnp.float32)
    h += jnp.dot(hi, W1_ref[Hh:, :], preferred_element_type=jnp.float32)
    h += b1_ref[...]
    h_ref[...] = h

    @pl.when(i == 0)
    def _():
        acc_ref[...] = jnp.zeros_like(acc_ref)

    acc_ref[0:1, :] += jnp.sum(h, axis=0, keepdims=True)
    acc_ref[1:2, :] += jnp.sum(h * h, axis=0, keepdims=True)

    @pl.when(i == GB - 1)
    def _():
        stats_ref[...] = acc_ref[...]


def _mlp1(x2, agg2, W1, b1):
    Hh = x2.shape[2]
    return pl.pallas_call(
        _mlp1_body,
        grid=(GB,),
        in_specs=[
            pl.BlockSpec((2, R_BLK, Hh), lambda i: (0, i, 0)),
            pl.BlockSpec((2, R_BLK, Hh), lambda i: (0, i, 0)),
            pl.BlockSpec(W1.shape, lambda i: (0, 0)),
            pl.BlockSpec((1, HID), lambda i: (0, 0)),
        ],
        out_specs=[
            pl.BlockSpec((R_BLK, HID), lambda i: (i, 0)),
            pl.BlockSpec((8, HID), lambda i: (0, 0)),
        ],
        out_shape=[
            jax.ShapeDtypeStruct((N, HID), jnp.float32),
            jax.ShapeDtypeStruct((8, HID), jnp.float32),
        ],
        scratch_shapes=[pltpu.VMEM((8, HID), jnp.float32)],
    )(x2, agg2, W1, b1.reshape(1, HID))


def _mlp2_body(h_ref, stats_ref, g_ref, be_ref, W2_ref, b2_ref, out_ref):
    mu = stats_ref[0:1, :] * (1.0 / N)
    ex2 = stats_ref[1:2, :] * (1.0 / N)
    var = ex2 - mu * mu
    scale = g_ref[...] * lax.rsqrt(var + 1e-5)
    hb = jnp.maximum((h_ref[...] - mu) * scale + be_ref[...], 0.0)
    o = jnp.dot(hb, W2_ref[...], preferred_element_type=jnp.float32)
    o = jnp.maximum(o + b2_ref[...], 0.0)
    out_ref[0] = o[:, :HID // 2]
    out_ref[1] = o[:, HID // 2:]


def _mlp2(h, stats, g, be, W2, b2):
    return pl.pallas_call(
        _mlp2_body,
        grid=(GB,),
        in_specs=[
            pl.BlockSpec((R_BLK, HID), lambda i: (i, 0)),
            pl.BlockSpec((8, HID), lambda i: (0, 0)),
            pl.BlockSpec((1, HID), lambda i: (0, 0)),
            pl.BlockSpec((1, HID), lambda i: (0, 0)),
            pl.BlockSpec((HID, HID), lambda i: (0, 0)),
            pl.BlockSpec((1, HID), lambda i: (0, 0)),
        ],
        out_specs=pl.BlockSpec((2, R_BLK, HID // 2), lambda i: (0, i, 0)),
        out_shape=jax.ShapeDtypeStruct((2, N, HID // 2), jnp.float32),
    )(h, stats, g.reshape(1, HID), be.reshape(1, HID), W2, b2.reshape(1, HID))


def _final_body(b_ref, x3_ref, W1_ref, b1_ref, W2_ref, b2_ref, out_ref, pool_ref):
    i = pl.program_id(0)

    @pl.when(i == 0)
    def _():
        pool_ref[...] = jnp.zeros_like(pool_ref)

    bv = jnp.minimum(b_ref[0], B_GRAPHS - 1)
    oh = (lax.broadcasted_iota(jnp.int32, (B_GRAPHS, R_BLK), 0)
          == jnp.broadcast_to(bv, (B_GRAPHS, R_BLK))).astype(jnp.float32)
    pool_ref[:, :HID // 2] += jnp.dot(oh, x3_ref[0], preferred_element_type=jnp.float32)
    pool_ref[:, HID // 2:] += jnp.dot(oh, x3_ref[1], preferred_element_type=jnp.float32)

    @pl.when(i == GB - 1)
    def _():
        hh = jnp.dot(pool_ref[...], W1_ref[...], preferred_element_type=jnp.float32)
        hh = jnp.maximum(hh + b1_ref[...], 0.0)
        out_ref[...] = jnp.dot(hh, W2_ref[...], preferred_element_type=jnp.float32) + b2_ref[...]


def _final(batch2d, x3, f_W1, f_b1, f_W2, f_b2):
    return pl.pallas_call(
        _final_body,
        grid=(GB,),
        in_specs=[
            pl.BlockSpec((1, 1, R_BLK), lambda i: (i, 0, 0)),
            pl.BlockSpec((2, R_BLK, HID // 2), lambda i: (0, i, 0)),
            pl.BlockSpec((HID, HID), lambda i: (0, 0)),
            pl.BlockSpec((1, HID), lambda i: (0, 0)),
            pl.BlockSpec((HID, OUT), lambda i: (0, 0)),
            pl.BlockSpec((1, OUT), lambda i: (0, 0)),
        ],
        out_specs=pl.BlockSpec((B_GRAPHS, OUT), lambda i: (0, 0)),
        out_shape=jax.ShapeDtypeStruct((B_GRAPHS, OUT), jnp.float32),
        scratch_shapes=[pltpu.VMEM((B_GRAPHS, HID), jnp.float32)],
    )(batch2d, x3, f_W1, f_b1.reshape(1, HID), f_W2, f_b2.reshape(1, OUT))


def kernel(x, edge_index, batch, batch_size,
           c0_W1, c0_b1, c0_g, c0_be, c0_W2, c0_b2,
           c1_W1, c1_b1, c1_g, c1_be, c1_W2, c1_b2,
           c2_W1, c2_b1, c2_g, c2_be, c2_W2, c2_b2,
           f_W1, f_b1, f_W2, f_b2):
    src = edge_index[0]
    dst = edge_index[1]
    e = src.shape[0]
    pad = E_PAD - e
    srcp = jnp.concatenate([src, jnp.zeros((pad,), jnp.int32)])
    dstp = jnp.concatenate([dst, jnp.full((pad,), N, jnp.int32)])
    srcg = jnp.stack([srcp, srcp + N]).reshape(2, CHUNKS, CH)
    src2d = srcp.reshape(CHUNKS, CH)
    dstg = dstp.reshape(CHUNKS, CH)
    z128 = jnp.zeros((CH, 128), jnp.float32)

    agg0 = _get_sc_agg(128, True)(x, src2d, dstg, z128)
    h0, st0 = _mlp1a(x, agg0, c0_W1, c0_b1)
    x1 = _mlp2(h0, st0, c0_g, c0_be, c0_W2, c0_b2)

    agg1 = _get_sc_agg(128, False)(x1.reshape(2 * N, HID // 2), srcg, dstg, z128)
    h1, st1 = _mlp1(x1, agg1, c1_W1, c1_b1)
    x2_ = _mlp2(h1, st1, c1_g, c1_be, c1_W2, c1_b2)

    agg2 = _get_sc_agg(128, False)(x2_.reshape(2 * N, HID // 2), srcg, dstg, z128)
    h2, st2 = _mlp1(x2_, agg2, c2_W1, c2_b1)
    x3 = _mlp2(h2, st2, c2_g, c2_be, c2_W2, c2_b2)

    return _final(batch.reshape(GB, 1, R_BLK), x3, f_W1, f_b1, f_W2, f_b2)


# speedup vs baseline: 3.0241x; 1.0530x over previous
import functools

import jax
import jax.numpy as jnp
from jax import lax
from jax.experimental import pallas as pl
from jax.experimental.pallas import tpu as pltpu
from jax.experimental.pallas import tpu_sc as plsc

N = 10000
B_GRAPHS = 64
HID = 256
OUT = 128

CH = 64
NSUB = 16
NBUF = 4
CPS = 320
SI = CPS // NBUF
CHUNKS = CPS * NSUB
E_PAD = CHUNKS * CH
RZ = 640
N_ACC = NSUB * RZ

R_BLK = 1000
GB = N // R_BLK


def _make_sc_agg(Hh, edge_split):
    mesh = plsc.VectorSubcoreMesh(core_axis_name="c", subcore_axis_name="s")

    def body(x2f, sdg, zrows, out, acc, sb, rows,
             g0, g1, g2, g3, t0, t1, t2, t3, i0, i1):
        c = lax.axis_index("c")
        s = lax.axis_index("s")
        gs = (g0, g1, g2, g3)
        ts = (t0, t1, t2, t3)
        isems = (i0, i1)

        pltpu.sync_copy(zrows, rows.at[0])
        for k in range(RZ // CH):
            pltpu.sync_copy(rows.at[0], acc.at[pl.ds(s * RZ + k * CH, CH)])
        plsc.subcore_barrier()

        if edge_split:
            cbase = (c * NSUB + s) * (CPS // 2)
            iters = SI // 2
        else:
            cbase = s * CPS
            iters = SI

        def slab_src(gi):
            if edge_split:
                return sdg.at[pl.ds(cbase + gi * NBUF, NBUF)]
            return sdg.at[c, pl.ds(cbase + gi * NBUF, NBUF)]

        def gather(slab, b):
            return x2f.at[sb.at[slab, b, 0]], rows.at[b], gs[b]

        def scatter(slab, b):
            return rows.at[b], acc.at[sb.at[slab, b, 1]], ts[b]

        pltpu.sync_copy(slab_src(0), sb.at[0])
        for b in range(NBUF):
            pltpu.async_copy(*gather(0, b))

        def halfiter(gidx, slab, nslab):
            @pl.when(gidx + 1 < iters)
            def _():
                pltpu.async_copy(slab_src(gidx + 1), sb.at[nslab], isems[nslab])

            for b in range(NBUF):
                pltpu.make_async_copy(*gather(slab, b)).wait()
                pltpu.async_copy(*scatter(slab, b), add=True)

            @pl.when(gidx + 1 < iters)
            def _():
                pltpu.make_async_copy(slab_src(gidx + 1), sb.at[nslab],
                                      isems[nslab]).wait()

            for b in range(NBUF):
                pltpu.make_async_copy(*scatter(slab, b)).wait()

                @pl.when(gidx + 1 < iters)
                def _():
                    pltpu.async_copy(*gather(nslab, b))

        def pairbody(p, carry):
            halfiter(2 * p, 0, 1)
            halfiter(2 * p + 1, 1, 0)
            return carry

        lax.fori_loop(0, iters // 2, pairbody, 0)
        plsc.subcore_barrier()

        for k in range(RZ // CH):
            pltpu.sync_copy(acc.at[pl.ds(s * RZ + k * CH, CH)], rows.at[0])
            pltpu.sync_copy(rows.at[0], out.at[c, pl.ds(s * RZ + k * CH, CH)])

    return pl.kernel(
        body,
        out_type=jax.ShapeDtypeStruct((2, N_ACC, Hh), jnp.float32),
        mesh=mesh,
        scratch_types=[
            pltpu.VMEM_SHARED((N_ACC, Hh), jnp.float32),
            pltpu.VMEM((2, NBUF, 2, CH), jnp.int32),
            pltpu.VMEM((NBUF, CH, Hh), jnp.float32),
            pltpu.SemaphoreType.DMA,
            pltpu.SemaphoreType.DMA,
            pltpu.SemaphoreType.DMA,
            pltpu.SemaphoreType.DMA,
            pltpu.SemaphoreType.DMA,
            pltpu.SemaphoreType.DMA,
            pltpu.SemaphoreType.DMA,
            pltpu.SemaphoreType.DMA,
            pltpu.SemaphoreType.DMA,
            pltpu.SemaphoreType.DMA,
        ],
    )


@functools.lru_cache(maxsize=None)
def _get_sc_agg(Hh, edge_split):
    return _make_sc_agg(Hh, edge_split)


def _mlp1a_body(x_ref, agg_ref, W1_ref, b1_ref, h_ref, stats_ref, acc_ref):
    i = pl.program_id(0)
    xa = x_ref[...] + agg_ref[0] + agg_ref[1]
    h = jnp.dot(xa, W1_ref[...], preferred_element_type=jnp.float32)
    h += b1_ref[...]
    h_ref[...] = h

    @pl.when(i == 0)
    def _():
        acc_ref[...] = jnp.zeros_like(acc_ref)

    acc_ref[0:1, :] += jnp.sum(h, axis=0, keepdims=True)
    acc_ref[1:2, :] += jnp.sum(h * h, axis=0, keepdims=True)

    @pl.when(i == GB - 1)
    def _():
        stats_ref[...] = acc_ref[...]


def _mlp1a(x, agg2, W1, b1):
    d_in = x.shape[1]
    return pl.pallas_call(
        _mlp1a_body,
        grid=(GB,),
        in_specs=[
            pl.BlockSpec((R_BLK, d_in), lambda i: (i, 0)),
            pl.BlockSpec((2, R_BLK, d_in), lambda i: (0, i, 0)),
            pl.BlockSpec(W1.shape, lambda i: (0, 0)),
            pl.BlockSpec((1, HID), lambda i: (0, 0)),
        ],
        out_specs=[
            pl.BlockSpec((R_BLK, HID), lambda i: (i, 0)),
            pl.BlockSpec((8, HID), lambda i: (0, 0)),
        ],
        out_shape=[
            jax.ShapeDtypeStruct((N, HID), jnp.float32),
            jax.ShapeDtypeStruct((8, HID), jnp.float32),
        ],
        scratch_shapes=[pltpu.VMEM((8, HID), jnp.float32)],
    )(x, agg2, W1, b1.reshape(1, HID))


def _mlp1_body(x2_ref, agg_ref, W1_ref, b1_ref, h_ref, stats_ref, acc_ref):
    i = pl.program_id(0)
    Hh = x2_ref.shape[2]
    lo = x2_ref[0] + agg_ref[0]
    hi = x2_ref[1] + agg_ref[1]
    h = jnp.dot(lo, W1_ref[:Hh, :], preferred_element_type=jnp.float32)
    h += jnp.dot(hi, W1_ref[Hh:, :], preferred_element_type=jnp.float32)
    h += b1_ref[...]
    h_ref[...] = h

    @pl.when(i == 0)
    def _():
        acc_ref[...] = jnp.zeros_like(acc_ref)

    acc_ref[0:1, :] += jnp.sum(h, axis=0, keepdims=True)
    acc_ref[1:2, :] += jnp.sum(h * h, axis=0, keepdims=True)

    @pl.when(i == GB - 1)
    def _():
        stats_ref[...] = acc_ref[...]


def _mlp1(x2, agg2, W1, b1):
    Hh = x2.shape[2]
    return pl.pallas_call(
        _mlp1_body,
        grid=(GB,),
        in_specs=[
            pl.BlockSpec((2, R_BLK, Hh), lambda i: (0, i, 0)),
            pl.BlockSpec((2, R_BLK, Hh), lambda i: (0, i, 0)),
            pl.BlockSpec(W1.shape, lambda i: (0, 0)),
            pl.BlockSpec((1, HID), lambda i: (0, 0)),
        ],
        out_specs=[
            pl.BlockSpec((R_BLK, HID), lambda i: (i, 0)),
            pl.BlockSpec((8, HID), lambda i: (0, 0)),
        ],
        out_shape=[
            jax.ShapeDtypeStruct((N, HID), jnp.float32),
            jax.ShapeDtypeStruct((8, HID), jnp.float32),
        ],
        scratch_shapes=[pltpu.VMEM((8, HID), jnp.float32)],
    )(x2, agg2, W1, b1.reshape(1, HID))


def _mlp2_body(h_ref, stats_ref, g_ref, be_ref, W2_ref, b2_ref, out_ref):
    mu = stats_ref[0:1, :] * (1.0 / N)
    ex2 = stats_ref[1:2, :] * (1.0 / N)
    var = ex2 - mu * mu
    scale = g_ref[...] * lax.rsqrt(var + 1e-5)
    hb = jnp.maximum((h_ref[...] - mu) * scale + be_ref[...], 0.0)
    o = jnp.dot(hb, W2_ref[...], preferred_element_type=jnp.float32)
    o = jnp.maximum(o + b2_ref[...], 0.0)
    out_ref[0] = o[:, :HID // 2]
    out_ref[1] = o[:, HID // 2:]


def _mlp2(h, stats, g, be, W2, b2):
    return pl.pallas_call(
        _mlp2_body,
        grid=(GB,),
        in_specs=[
            pl.BlockSpec((R_BLK, HID), lambda i: (i, 0)),
            pl.BlockSpec((8, HID), lambda i: (0, 0)),
            pl.BlockSpec((1, HID), lambda i: (0, 0)),
            pl.BlockSpec((1, HID), lambda i: (0, 0)),
            pl.BlockSpec((HID, HID), lambda i: (0, 0)),
            pl.BlockSpec((1, HID), lambda i: (0, 0)),
        ],
        out_specs=pl.BlockSpec((2, R_BLK, HID // 2), lambda i: (0, i, 0)),
        out_shape=jax.ShapeDtypeStruct((2, N, HID // 2), jnp.float32),
    )(h, stats, g.reshape(1, HID), be.reshape(1, HID), W2, b2.reshape(1, HID))


def _final_body(b_ref, x3_ref, W1_ref, b1_ref, W2_ref, b2_ref, out_ref, pool_ref):
    i = pl.program_id(0)

    @pl.when(i == 0)
    def _():
        pool_ref[...] = jnp.zeros_like(pool_ref)

    bv = jnp.minimum(b_ref[0], B_GRAPHS - 1)
    oh = (lax.broadcasted_iota(jnp.int32, (B_GRAPHS, R_BLK), 0)
          == jnp.broadcast_to(bv, (B_GRAPHS, R_BLK))).astype(jnp.float32)
    pool_ref[:, :HID // 2] += jnp.dot(oh, x3_ref[0], preferred_element_type=jnp.float32)
    pool_ref[:, HID // 2:] += jnp.dot(oh, x3_ref[1], preferred_element_type=jnp.float32)

    @pl.when(i == GB - 1)
    def _():
        hh = jnp.dot(pool_ref[...], W1_ref[...], preferred_element_type=jnp.float32)
        hh = jnp.maximum(hh + b1_ref[...], 0.0)
        out_ref[...] = jnp.dot(hh, W2_ref[...], preferred_element_type=jnp.float32) + b2_ref[...]


def _final(batch2d, x3, f_W1, f_b1, f_W2, f_b2):
    return pl.pallas_call(
        _final_body,
        grid=(GB,),
        in_specs=[
            pl.BlockSpec((1, 1, R_BLK), lambda i: (i, 0, 0)),
            pl.BlockSpec((2, R_BLK, HID // 2), lambda i: (0, i, 0)),
            pl.BlockSpec((HID, HID), lambda i: (0, 0)),
            pl.BlockSpec((1, HID), lambda i: (0, 0)),
            pl.BlockSpec((HID, OUT), lambda i: (0, 0)),
            pl.BlockSpec((1, OUT), lambda i: (0, 0)),
        ],
        out_specs=pl.BlockSpec((B_GRAPHS, OUT), lambda i: (0, 0)),
        out_shape=jax.ShapeDtypeStruct((B_GRAPHS, OUT), jnp.float32),
        scratch_shapes=[pltpu.VMEM((B_GRAPHS, HID), jnp.float32)],
    )(batch2d, x3, f_W1, f_b1.reshape(1, HID), f_W2, f_b2.reshape(1, OUT))


def kernel(x, edge_index, batch, batch_size,
           c0_W1, c0_b1, c0_g, c0_be, c0_W2, c0_b2,
           c1_W1, c1_b1, c1_g, c1_be, c1_W2, c1_b2,
           c2_W1, c2_b1, c2_g, c2_be, c2_W2, c2_b2,
           f_W1, f_b1, f_W2, f_b2):
    src = edge_index[0]
    dst = edge_index[1]
    e = src.shape[0]
    pad = E_PAD - e
    srcp = jnp.concatenate([src, jnp.zeros((pad,), jnp.int32)])
    dstp = jnp.concatenate([dst, jnp.full((pad,), N, jnp.int32)])
    srcr = srcp.reshape(CHUNKS, CH)
    dstr = dstp.reshape(CHUNKS, CH)
    sdg0 = jnp.stack([srcr, dstr], 1)
    sdg2 = jnp.stack([sdg0, jnp.stack([srcr + N, dstr], 1)])
    z128 = jnp.zeros((CH, 128), jnp.float32)

    agg0 = _get_sc_agg(128, True)(x, sdg0, z128)
    h0, st0 = _mlp1a(x, agg0, c0_W1, c0_b1)
    x1 = _mlp2(h0, st0, c0_g, c0_be, c0_W2, c0_b2)

    agg1 = _get_sc_agg(128, False)(x1.reshape(2 * N, HID // 2), sdg2, z128)
    h1, st1 = _mlp1(x1, agg1, c1_W1, c1_b1)
    x2_ = _mlp2(h1, st1, c1_g, c1_be, c1_W2, c1_b2)

    agg2 = _get_sc_agg(128, False)(x2_.reshape(2 * N, HID // 2), sdg2, z128)
    h2, st2 = _mlp1(x2_, agg2, c2_W1, c2_b1)
    x3 = _mlp2(h2, st2, c2_g, c2_be, c2_W2, c2_b2)

    return _final(batch.reshape(GB, 1, R_BLK), x3, f_W1, f_b1, f_W2, f_b2)


# speedup vs baseline: 5.2203x; 1.7262x over previous
import functools

import jax
import jax.numpy as jnp
from jax import lax
from jax.experimental import pallas as pl
from jax.experimental.pallas import tpu as pltpu
from jax.experimental.pallas import tpu_sc as plsc

N = 10000
B_GRAPHS = 64
HID = 256
OUT = 128
QW = 64

CH = 64
NSUB = 16
NBUF = 4
CPS = 320
SI = CPS // NBUF
CHUNKS = CPS * NSUB
E_PAD = CHUNKS * CH
RZ = 640
N_ACC = NSUB * RZ
RSTRIPS = RZ // CH

R_BLK = 1000
GB = N // R_BLK


def _make_sc_agg(nq):
    mesh = plsc.VectorSubcoreMesh(core_axis_name="c", subcore_axis_name="s")

    def body(xq, sdg, zrows, out, xs, acc, sb, rows,
             g0, g1, g2, g3, t0, t1, t2, t3, i0, i1):
        c = lax.axis_index("c")
        s = lax.axis_index("s")
        gs = (g0, g1, g2, g3)
        ts = (t0, t1, t2, t3)
        isems = (i0, i1)

        def slab_src(gi):
            return sdg.at[pl.ds(s * CPS + gi * NBUF, NBUF)]

        def gather(slab, b):
            return xs.at[sb.at[slab, b, 0]], rows.at[b], gs[b]

        def scatter(slab, b):
            return rows.at[b], acc.at[sb.at[slab, b, 1]], ts[b]

        for q in range(nq // 2):
            qi = c * (nq // 2) + q

            pltpu.sync_copy(zrows, rows.at[0])
            for k in range(RSTRIPS):
                r0 = s * RZ + k * CH
                pltpu.sync_copy(xq.at[qi, pl.ds(r0, CH)], xs.at[pl.ds(r0, CH)])
                pltpu.sync_copy(rows.at[0], acc.at[pl.ds(r0, CH)])
            plsc.subcore_barrier()

            pltpu.sync_copy(slab_src(0), sb.at[0])
            for b in range(NBUF):
                pltpu.async_copy(*gather(0, b))

            def halfiter(gidx, slab, nslab):
                @pl.when(gidx + 1 < SI)
                def _():
                    pltpu.async_copy(slab_src(gidx + 1), sb.at[nslab],
                                     isems[nslab])

                for b in range(NBUF):
                    pltpu.make_async_copy(*gather(slab, b)).wait()
                    pltpu.async_copy(*scatter(slab, b), add=True)

                @pl.when(gidx + 1 < SI)
                def _():
                    pltpu.make_async_copy(slab_src(gidx + 1), sb.at[nslab],
                                          isems[nslab]).wait()

                for b in range(NBUF):
                    pltpu.make_async_copy(*scatter(slab, b)).wait()

                    @pl.when(gidx + 1 < SI)
                    def _():
                        pltpu.async_copy(*gather(nslab, b))

            def pairbody(p, carry):
                halfiter(2 * p, 0, 1)
                halfiter(2 * p + 1, 1, 0)
                return carry

            lax.fori_loop(0, SI // 2, pairbody, 0)
            plsc.subcore_barrier()

            for k in range(RSTRIPS):
                r0 = s * RZ + k * CH
                pltpu.sync_copy(acc.at[pl.ds(r0, CH)], rows.at[0])
                pltpu.sync_copy(rows.at[0], out.at[qi, pl.ds(r0, CH)])
            if q + 1 < nq // 2:
                plsc.subcore_barrier()

    return pl.kernel(
        body,
        out_type=jax.ShapeDtypeStruct((nq, N_ACC, QW), jnp.float32),
        mesh=mesh,
        scratch_types=[
            pltpu.VMEM_SHARED((N_ACC, QW), jnp.float32),
            pltpu.VMEM_SHARED((N_ACC, QW), jnp.float32),
            pltpu.VMEM((2, NBUF, 2, CH), jnp.int32),
            pltpu.VMEM((NBUF, CH, QW), jnp.float32),
            pltpu.SemaphoreType.DMA,
            pltpu.SemaphoreType.DMA,
            pltpu.SemaphoreType.DMA,
            pltpu.SemaphoreType.DMA,
            pltpu.SemaphoreType.DMA,
            pltpu.SemaphoreType.DMA,
            pltpu.SemaphoreType.DMA,
            pltpu.SemaphoreType.DMA,
            pltpu.SemaphoreType.DMA,
            pltpu.SemaphoreType.DMA,
        ],
    )


@functools.lru_cache(maxsize=None)
def _get_sc_agg(nq):
    return _make_sc_agg(nq)


def _mlp1_body(xq_ref, agg_ref, W1_ref, b1_ref, h_ref, stats_ref, acc_ref):
    i = pl.program_id(0)
    nq = xq_ref.shape[0]
    h = b1_ref[...]
    for q in range(nq):
        h += jnp.dot(xq_ref[q] + agg_ref[q], W1_ref[q * QW:(q + 1) * QW, :],
                     preferred_element_type=jnp.float32)
    h_ref[...] = h

    @pl.when(i == 0)
    def _():
        acc_ref[...] = jnp.zeros_like(acc_ref)

    acc_ref[0:1, :] += jnp.sum(h, axis=0, keepdims=True)
    acc_ref[1:2, :] += jnp.sum(h * h, axis=0, keepdims=True)

    @pl.when(i == GB - 1)
    def _():
        stats_ref[...] = acc_ref[...]


def _mlp1(xq, agg, W1, b1):
    nq = xq.shape[0]
    return pl.pallas_call(
        _mlp1_body,
        grid=(GB,),
        in_specs=[
            pl.BlockSpec((nq, R_BLK, QW), lambda i: (0, i, 0)),
            pl.BlockSpec((nq, R_BLK, QW), lambda i: (0, i, 0)),
            pl.BlockSpec(W1.shape, lambda i: (0, 0)),
            pl.BlockSpec((1, HID), lambda i: (0, 0)),
        ],
        out_specs=[
            pl.BlockSpec((R_BLK, HID), lambda i: (i, 0)),
            pl.BlockSpec((8, HID), lambda i: (0, 0)),
        ],
        out_shape=[
            jax.ShapeDtypeStruct((N, HID), jnp.float32),
            jax.ShapeDtypeStruct((8, HID), jnp.float32),
        ],
        scratch_shapes=[pltpu.VMEM((8, HID), jnp.float32)],
    )(xq, agg, W1, b1.reshape(1, HID))


def _mlp2_body(h_ref, stats_ref, g_ref, be_ref, W2_ref, b2_ref, out_ref):
    mu = stats_ref[0:1, :] * (1.0 / N)
    ex2 = stats_ref[1:2, :] * (1.0 / N)
    var = ex2 - mu * mu
    scale = g_ref[...] * lax.rsqrt(var + 1e-5)
    hb = jnp.maximum((h_ref[...] - mu) * scale + be_ref[...], 0.0)
    o = jnp.dot(hb, W2_ref[...], preferred_element_type=jnp.float32)
    o = jnp.maximum(o + b2_ref[...], 0.0)
    for q in range(4):
        out_ref[q] = o[:, q * QW:(q + 1) * QW]


def _mlp2(h, stats, g, be, W2, b2):
    return pl.pallas_call(
        _mlp2_body,
        grid=(GB,),
        in_specs=[
            pl.BlockSpec((R_BLK, HID), lambda i: (i, 0)),
            pl.BlockSpec((8, HID), lambda i: (0, 0)),
            pl.BlockSpec((1, HID), lambda i: (0, 0)),
            pl.BlockSpec((1, HID), lambda i: (0, 0)),
            pl.BlockSpec((HID, HID), lambda i: (0, 0)),
            pl.BlockSpec((1, HID), lambda i: (0, 0)),
        ],
        out_specs=pl.BlockSpec((4, R_BLK, QW), lambda i: (0, i, 0)),
        out_shape=jax.ShapeDtypeStruct((4, N_ACC, QW), jnp.float32),
    )(h, stats, g.reshape(1, HID), be.reshape(1, HID), W2, b2.reshape(1, HID))


def _final_body(b_ref, x3_ref, W1_ref, b1_ref, W2_ref, b2_ref, out_ref, pool_ref):
    i = pl.program_id(0)

    @pl.when(i == 0)
    def _():
        pool_ref[...] = jnp.zeros_like(pool_ref)

    bv = jnp.minimum(b_ref[0], B_GRAPHS - 1)
    oh = (lax.broadcasted_iota(jnp.int32, (B_GRAPHS, R_BLK), 0)
          == jnp.broadcast_to(bv, (B_GRAPHS, R_BLK))).astype(jnp.float32)
    for q in range(4):
        pool_ref[:, q * QW:(q + 1) * QW] += jnp.dot(
            oh, x3_ref[q], preferred_element_type=jnp.float32)

    @pl.when(i == GB - 1)
    def _():
        hh = jnp.dot(pool_ref[...], W1_ref[...], preferred_element_type=jnp.float32)
        hh = jnp.maximum(hh + b1_ref[...], 0.0)
        out_ref[...] = jnp.dot(hh, W2_ref[...], preferred_element_type=jnp.float32) + b2_ref[...]


def _final(batch3d, x3, f_W1, f_b1, f_W2, f_b2):
    return pl.pallas_call(
        _final_body,
        grid=(GB,),
        in_specs=[
            pl.BlockSpec((1, 1, R_BLK), lambda i: (i, 0, 0)),
            pl.BlockSpec((4, R_BLK, QW), lambda i: (0, i, 0)),
            pl.BlockSpec((HID, HID), lambda i: (0, 0)),
            pl.BlockSpec((1, HID), lambda i: (0, 0)),
            pl.BlockSpec((HID, OUT), lambda i: (0, 0)),
            pl.BlockSpec((1, OUT), lambda i: (0, 0)),
        ],
        out_specs=pl.BlockSpec((B_GRAPHS, OUT), lambda i: (0, 0)),
        out_shape=jax.ShapeDtypeStruct((B_GRAPHS, OUT), jnp.float32),
        scratch_shapes=[pltpu.VMEM((B_GRAPHS, HID), jnp.float32)],
    )(batch3d, x3, f_W1, f_b1.reshape(1, HID), f_W2, f_b2.reshape(1, OUT))


def kernel(x, edge_index, batch, batch_size,
           c0_W1, c0_b1, c0_g, c0_be, c0_W2, c0_b2,
           c1_W1, c1_b1, c1_g, c1_be, c1_W2, c1_b2,
           c2_W1, c2_b1, c2_g, c2_be, c2_W2, c2_b2,
           f_W1, f_b1, f_W2, f_b2):
    src = edge_index[0]
    dst = edge_index[1]
    e = src.shape[0]
    pad = E_PAD - e
    srcp = jnp.concatenate([src, jnp.zeros((pad,), jnp.int32)])
    dstp = jnp.concatenate([dst, jnp.full((pad,), N, jnp.int32)])
    sdg = jnp.stack([srcp.reshape(CHUNKS, CH), dstp.reshape(CHUNKS, CH)], 1)
    z = jnp.zeros((CH, QW), jnp.float32)

    d_in = x.shape[1]
    nq0 = d_in // QW
    xq0 = jnp.zeros((nq0, N_ACC, QW), jnp.float32).at[:, :N, :].set(
        jnp.moveaxis(x.reshape(N, nq0, QW), 1, 0))

    agg0 = _get_sc_agg(nq0)(xq0, sdg, z)
    h0, st0 = _mlp1(xq0, agg0, c0_W1, c0_b1)
    x1 = _mlp2(h0, st0, c0_g, c0_be, c0_W2, c0_b2)

    agg1 = _get_sc_agg(4)(x1, sdg, z)
    h1, st1 = _mlp1(x1, agg1, c1_W1, c1_b1)
    x2_ = _mlp2(h1, st1, c1_g, c1_be, c1_W2, c1_b2)

    agg2 = _get_sc_agg(4)(x2_, sdg, z)
    h2, st2 = _mlp1(x2_, agg2, c2_W1, c2_b1)
    x3 = _mlp2(h2, st2, c2_g, c2_be, c2_W2, c2_b2)

    return _final(batch.reshape(GB, 1, R_BLK), x3, f_W1, f_b1, f_W2, f_b2)
